# baseline (speedup 1.0000x reference)
import functools

import jax
import jax.numpy as jnp
from jax import lax
from jax.experimental import pallas as pl
from jax.experimental.pallas import tpu as pltpu
from jax.experimental.pallas import tpu_sc as plsc

_NC = 2
_NS = 16
_NW = _NC * _NS


@functools.lru_cache(maxsize=None)
def _build(nb, h, d, n_out_rows):
    rows_w = nb // _NW
    k_grp = 8
    group = k_grp * h
    ng = rows_w // k_grp
    mesh = plsc.VectorSubcoreMesh(core_axis_name="c", subcore_axis_name="s")

    @functools.partial(
        pl.kernel,
        out_type=jax.ShapeDtypeStruct((n_out_rows, d), jnp.float32),
        mesh=mesh,
        compiler_params=pltpu.CompilerParams(use_tc_tiling_on_sc=False),
        scratch_types=[
            pltpu.VMEM((rows_w, h), jnp.int32),
            pltpu.VMEM((rows_w, h), jnp.int32),
            pltpu.VMEM((group, d), jnp.float32),
            pltpu.VMEM((group, d), jnp.float32),
            pltpu.SemaphoreType.DMA,
            pltpu.SemaphoreType.DMA,
            pltpu.SemaphoreType.DMA,
        ],
    )
    def grab(x_hbm, srow_hbm, table_hbm, out_hbm, idx_v, sidx_v,
             rows0, rows1, gsem, wsem0, wsem1):
        wid = lax.axis_index("s") * _NC + lax.axis_index("c")
        base = wid * rows_w
        pltpu.sync_copy(x_hbm.at[pl.ds(base, rows_w)], idx_v)
        pltpu.sync_copy(srow_hbm.at[pl.ds(base, rows_w)], sidx_v)

        rows = (rows0, rows1)
        wsem = (wsem0, wsem1)

        def slot(g, b, guard):
            @pl.when(guard)
            def _():
                pltpu.make_async_copy(
                    rows[b], out_hbm.at[pl.ds(0, group)], wsem[b]
                ).wait()

            descs = []
            for c in range(k_grp):
                descs.append(
                    pltpu.async_copy(
                        table_hbm.at[idx_v.at[g * k_grp + c]],
                        rows[b].at[pl.ds(c * h, h)],
                        gsem,
                    )
                )
            for c in range(k_grp):
                descs[c].wait()
                pltpu.async_copy(
                    rows[b].at[pl.ds(c * h, h)],
                    out_hbm.at[sidx_v.at[g * k_grp + c]],
                    wsem[b],
                )

        def body(g2, carry):
            slot(2 * g2, 0, g2 > 0)
            slot(2 * g2 + 1, 1, g2 > 0)
            return carry

        lax.fori_loop(0, ng // 2, body, 0)

        pltpu.make_async_copy(
            rows0, out_hbm.at[pl.ds(0, group)], wsem0
        ).wait()
        pltpu.make_async_copy(
            rows1, out_hbm.at[pl.ds(0, group)], wsem1
        ).wait()

    return grab


@jax.jit
def kernel(x, table):
    nb, h = x.shape
    d = table.shape[1]
    hpad = ((h + 7) // 8) * 8
    dpad = 128
    sub = dpad // d
    srow = sub * (
        hpad * jnp.arange(nb, dtype=jnp.int32)[:, None]
        + jnp.arange(h, dtype=jnp.int32)[None, :]
    )
    out = _build(nb, h, d, nb * hpad * sub)(
        x.astype(jnp.int32), srow, table
    )
    return out.reshape(nb, hpad, dpad)[:, :h, :d]


# speedup vs baseline: 1.0108x; 1.0108x over previous
import functools

import jax
import jax.numpy as jnp
from jax import lax
from jax.experimental import pallas as pl
from jax.experimental.pallas import tpu as pltpu
from jax.experimental.pallas import tpu_sc as plsc

_NC = 2
_NS = 16
_NW = _NC * _NS
_CHUNK = 128


@functools.lru_cache(maxsize=None)
def _build(n, d, n_out_rows):
    per_w = n // _NW
    nch = per_w // _CHUNK
    k_grp = 5
    group = k_grp * _CHUNK
    ng = per_w // group
    mesh = plsc.VectorSubcoreMesh(core_axis_name="c", subcore_axis_name="s")

    @functools.partial(
        pl.kernel,
        out_type=jax.ShapeDtypeStruct((n_out_rows, d), jnp.float32),
        mesh=mesh,
        compiler_params=pltpu.CompilerParams(use_tc_tiling_on_sc=False),
        scratch_types=[
            pltpu.VMEM((2 * nch, _CHUNK), jnp.int32),
            pltpu.VMEM((group, d), jnp.float32),
            pltpu.VMEM((group, d), jnp.float32),
            pltpu.SemaphoreType.DMA,
            pltpu.SemaphoreType.DMA,
            pltpu.SemaphoreType.DMA,
        ],
    )
    def grab(meta_hbm, table_hbm, out_hbm, meta_v,
             rows0, rows1, gsem, wsem0, wsem1):
        wid = lax.axis_index("s") * _NC + lax.axis_index("c")
        pltpu.sync_copy(meta_hbm.at[pl.ds(wid * 2 * nch, 2 * nch)], meta_v)

        rows = (rows0, rows1)
        wsem = (wsem0, wsem1)

        def slot(g, b, first):
            if not first:
                pltpu.make_async_copy(
                    rows[b], out_hbm.at[pl.ds(0, group)], wsem[b]
                ).wait()
            descs = []
            for c in range(k_grp):
                descs.append(
                    pltpu.async_copy(
                        table_hbm.at[meta_v.at[g * k_grp + c]],
                        rows[b].at[pl.ds(c * _CHUNK, _CHUNK)],
                        gsem,
                    )
                )
            for c in range(k_grp):
                descs[c].wait()
                pltpu.async_copy(
                    rows[b].at[pl.ds(c * _CHUNK, _CHUNK)],
                    out_hbm.at[meta_v.at[nch + g * k_grp + c]],
                    wsem[b],
                )

        slot(0, 0, True)
        slot(1, 1, True)

        def body(g2, carry):
            slot(2 * g2, 0, False)
            slot(2 * g2 + 1, 1, False)
            return carry

        lax.fori_loop(1, ng // 2, body, 0)

        pltpu.make_async_copy(
            rows0, out_hbm.at[pl.ds(0, group)], wsem0
        ).wait()
        pltpu.make_async_copy(
            rows1, out_hbm.at[pl.ds(0, group)], wsem1
        ).wait()

    return grab


@jax.jit
def kernel(x, table):
    b, h = x.shape
    d = table.shape[1]
    n = b * h
    hpad = ((h + 7) // 8) * 8
    dpad = 128
    sub = dpad // d
    per_w = n // _NW
    nch = per_w // _CHUNK
    idx = x.reshape(-1).astype(jnp.int32)
    r = jnp.arange(n, dtype=jnp.int32)
    srow = sub * ((r // h) * hpad + (r % h))
    meta = jnp.concatenate(
        [idx.reshape(_NW, nch, _CHUNK), srow.reshape(_NW, nch, _CHUNK)],
        axis=1,
    ).reshape(-1, _CHUNK)
    out = _build(n, d, b * hpad * sub)(meta, table)
    return out.reshape(b, hpad, dpad)[:, :h, :d]
